# baseline (speedup 1.0000x reference)
import functools

import jax
import jax.numpy as jnp
from jax import lax
from jax.experimental import pallas as pl
from jax.experimental.pallas import tpu as pltpu
from jax.experimental.pallas import tpu_sc as plsc

N_NODES = 10000
D = 128
N_EDGES = 320000

NC = 2
NS = 16
NW = NC * NS
EPW = N_EDGES // NW
CHUNK = 80
E_PAD = 327680
EPT = E_PAD // NW
NCHUNKS = EPT // CHUNK
NBUF = 4
NPAD = 10240
ROWS_PER_TILE = NPAD // NS

ROW_BLK = 1000
N_BLK = N_NODES // ROW_BLK


def _sc_edge_aggregate(x, src, dst, ew, zeros):
  mesh = plsc.VectorSubcoreMesh(core_axis_name="c", subcore_axis_name="s")

  @functools.partial(
      pl.kernel,
      mesh=mesh,
      out_type=jax.ShapeDtypeStruct((NC, NPAD, D), jnp.float32),
      scratch_types=(
          [pltpu.VMEM((CHUNK,), jnp.int32)] * NBUF
          + [pltpu.VMEM((CHUNK,), jnp.int32)] * NBUF
          + [pltpu.VMEM((CHUNK,), jnp.float32)] * NBUF
          + [pltpu.VMEM((CHUNK, D), jnp.float32)] * NBUF
          + [pltpu.VMEM_SHARED((NPAD, D), jnp.float32)]
          + [pltpu.SemaphoreType.DMA] * NBUF
      ),
  )
  def spmm(x_hbm, src_hbm, dst_hbm, ew_hbm, z_hbm, out_hbm, *refs):
    srcb = refs[0:NBUF]
    dstb = refs[NBUF:2 * NBUF]
    ewb = refs[2 * NBUF:3 * NBUF]
    rb = refs[3 * NBUF:4 * NBUF]
    acc_sh = refs[4 * NBUF]
    sems = refs[4 * NBUF + 1:]
    c = lax.axis_index("c")
    s = lax.axis_index("s")
    wid = c * NS + s

    pltpu.sync_copy(z_hbm.at[pl.ds(s * ROWS_PER_TILE, ROWS_PER_TILE)],
                    acc_sh.at[pl.ds(s * ROWS_PER_TILE, ROWS_PER_TILE)])
    plsc.subcore_barrier()

    def idx(i, sbuf, dbuf, wbuf):
      base = wid * EPT + i * CHUNK
      pltpu.sync_copy(src_hbm.at[pl.ds(base, CHUNK)], sbuf)
      pltpu.sync_copy(dst_hbm.at[pl.ds(base, CHUNK)], dbuf)
      pltpu.sync_copy(ew_hbm.at[pl.ds(base, CHUNK)], wbuf)

    def start_gather(sbuf, rbuf, sem):
      pltpu.async_copy(x_hbm.at[sbuf], rbuf, sem)

    def wait_gather(rbuf, sem):
      pltpu.make_async_copy(x_hbm.at[pl.ds(0, CHUNK)], rbuf, sem).wait()

    def process(wbuf, rbuf, dbuf):
      def scale_group(g, carry2):
        wv = wbuf[pl.ds(g * 16, 16)]
        for t in range(16):
          w = wv[t]
          j = g * 16 + t
          for q in range(D // 16):
            rbuf[j, pl.ds(q * 16, 16)] = rbuf[j, pl.ds(q * 16, 16)] * w
        return carry2

      lax.fori_loop(0, CHUNK // 16, scale_group, 0)
      pltpu.sync_copy(rbuf, acc_sh.at[dbuf], add=True)

    for b in range(NBUF):
      idx(b, srcb[b], dstb[b], ewb[b])
      start_gather(srcb[b], rb[b], sems[b])

    def body(q, carry):
      i0 = NBUF * q
      for b in range(NBUF):
        wait_gather(rb[b], sems[b])
        process(ewb[b], rb[b], dstb[b])
        idx(jnp.minimum(i0 + b + NBUF, NCHUNKS - 1), srcb[b], dstb[b], ewb[b])
        start_gather(srcb[b], rb[b], sems[b])
      return carry

    lax.fori_loop(0, NCHUNKS // NBUF, body, 0)
    for b in range(NBUF):
      wait_gather(rb[b], sems[b])

    plsc.subcore_barrier()
    pltpu.sync_copy(acc_sh.at[pl.ds(s * ROWS_PER_TILE, ROWS_PER_TILE)],
                    out_hbm.at[c, pl.ds(s * ROWS_PER_TILE, ROWS_PER_TILE)])

  return spmm(x, src, dst, ew, zeros)


def _tc_combine(agg, x, W, W_self, b):
  def kern(agg_ref, x_ref, w_ref, ws_ref, b_ref, out_ref, stats_ref,
           sum_acc, sq_acc):
    i = pl.program_id(0)
    a = agg_ref[0] + agg_ref[1]
    y = (lax.dot(a, w_ref[...], precision=lax.Precision.HIGHEST)
         + lax.dot(x_ref[...], ws_ref[...], precision=lax.Precision.HIGHEST)
         + b_ref[...])
    out_ref[...] = y

    @pl.when(i == 0)
    def _():
      sum_acc[...] = jnp.zeros_like(sum_acc)
      sq_acc[...] = jnp.zeros_like(sq_acc)

    sum_acc[...] += jnp.sum(y, axis=0, keepdims=True)
    sq_acc[...] += jnp.sum(y * y, axis=0, keepdims=True)

    @pl.when(i == N_BLK - 1)
    def _():
      stats_ref[0:1, :] = sum_acc[...]
      stats_ref[1:2, :] = sq_acc[...]

  return pl.pallas_call(
      kern,
      grid=(N_BLK,),
      in_specs=[
          pl.BlockSpec((NC, ROW_BLK, D), lambda i: (0, i, 0)),
          pl.BlockSpec((ROW_BLK, D), lambda i: (i, 0)),
          pl.BlockSpec((D, D), lambda i: (0, 0)),
          pl.BlockSpec((D, D), lambda i: (0, 0)),
          pl.BlockSpec((1, D), lambda i: (0, 0)),
      ],
      out_specs=[
          pl.BlockSpec((ROW_BLK, D), lambda i: (i, 0)),
          pl.BlockSpec((2, D), lambda i: (0, 0)),
      ],
      out_shape=[
          jax.ShapeDtypeStruct((N_NODES, D), jnp.float32),
          jax.ShapeDtypeStruct((2, D), jnp.float32),
      ],
      scratch_shapes=[
          pltpu.VMEM((1, D), jnp.float32),
          pltpu.VMEM((1, D), jnp.float32),
      ],
  )(agg, x, W, W_self, b)


def _tc_batchnorm(out_pre, stats, gamma, beta):
  def kern(y_ref, st_ref, g_ref, bt_ref, o_ref):
    mean = st_ref[0:1, :] * (1.0 / N_NODES)
    var = st_ref[1:2, :] * (1.0 / N_NODES) - mean * mean
    inv = lax.rsqrt(var + 1e-5) * g_ref[...]
    o_ref[...] = (y_ref[...] - mean) * inv + bt_ref[...]

  return pl.pallas_call(
      kern,
      grid=(N_BLK,),
      in_specs=[
          pl.BlockSpec((ROW_BLK, D), lambda i: (i, 0)),
          pl.BlockSpec((2, D), lambda i: (0, 0)),
          pl.BlockSpec((1, D), lambda i: (0, 0)),
          pl.BlockSpec((1, D), lambda i: (0, 0)),
      ],
      out_specs=pl.BlockSpec((ROW_BLK, D), lambda i: (i, 0)),
      out_shape=jax.ShapeDtypeStruct((N_NODES, D), jnp.float32),
  )(out_pre, stats, gamma, beta)


def kernel(x, edge_index, edge_weight, W, W_self, b, bn_gamma, bn_beta):
  ei = edge_index.astype(jnp.int32)
  pad = E_PAD - N_EDGES
  src = jnp.concatenate([ei[0], jnp.zeros((pad,), jnp.int32)])
  dst = jnp.concatenate([ei[1], jnp.zeros((pad,), jnp.int32)])
  edge_weight = jnp.concatenate([edge_weight, jnp.zeros((pad,), jnp.float32)])
  zeros = jnp.zeros((NPAD, D), jnp.float32)
  agg = _sc_edge_aggregate(x, src, dst, edge_weight, zeros)
  out_pre, stats = _tc_combine(agg, x, W, W_self, b.reshape(1, D))
  return _tc_batchnorm(out_pre, stats, bn_gamma.reshape(1, D),
                       bn_beta.reshape(1, D))


# speedup vs baseline: 1.0007x; 1.0007x over previous
import functools

import jax
import jax.numpy as jnp
from jax import lax
from jax.experimental import pallas as pl
from jax.experimental.pallas import tpu as pltpu
from jax.experimental.pallas import tpu_sc as plsc

N_NODES = 10000
D = 128
N_EDGES = 320000

NC = 2
NS = 16
NW = NC * NS
EPW = N_EDGES // NW
CHUNK = 80
E_PAD = 327680
EPT = E_PAD // NW
NCHUNKS = EPT // CHUNK
NBUF = 4
NPAD = 10240
ROWS_PER_TILE = NPAD // NS

ROW_BLK = 1000
N_BLK = N_NODES // ROW_BLK


def _sc_edge_aggregate(x, src, dst, ew, zeros):
  mesh = plsc.VectorSubcoreMesh(core_axis_name="c", subcore_axis_name="s")

  @functools.partial(
      pl.kernel,
      mesh=mesh,
      out_type=jax.ShapeDtypeStruct((NC, NPAD, D), jnp.float32),
      scratch_types=(
          [pltpu.VMEM((CHUNK,), jnp.int32)] * NBUF
          + [pltpu.VMEM((CHUNK,), jnp.int32)] * NBUF
          + [pltpu.VMEM((CHUNK,), jnp.float32)] * NBUF
          + [pltpu.VMEM((CHUNK, D), jnp.float32)] * NBUF
          + [pltpu.VMEM_SHARED((NPAD, D), jnp.float32)]
          + [pltpu.SemaphoreType.DMA] * NBUF
      ),
  )
  def spmm(x_hbm, src_hbm, dst_hbm, ew_hbm, z_hbm, out_hbm, *refs):
    srcb = refs[0:NBUF]
    dstb = refs[NBUF:2 * NBUF]
    ewb = refs[2 * NBUF:3 * NBUF]
    rb = refs[3 * NBUF:4 * NBUF]
    acc_sh = refs[4 * NBUF]
    sems = refs[4 * NBUF + 1:]
    c = lax.axis_index("c")
    s = lax.axis_index("s")
    wid = c * NS + s

    pltpu.sync_copy(z_hbm.at[pl.ds(s * ROWS_PER_TILE, ROWS_PER_TILE)],
                    acc_sh.at[pl.ds(s * ROWS_PER_TILE, ROWS_PER_TILE)])
    plsc.subcore_barrier()

    def idx(i, sbuf, dbuf, wbuf):
      base = wid * EPT + i * CHUNK
      pltpu.sync_copy(src_hbm.at[pl.ds(base, CHUNK)], sbuf)
      pltpu.sync_copy(dst_hbm.at[pl.ds(base, CHUNK)], dbuf)
      pltpu.sync_copy(ew_hbm.at[pl.ds(base, CHUNK)], wbuf)

    def start_gather(sbuf, rbuf, sem):
      pltpu.async_copy(x_hbm.at[sbuf], rbuf, sem)

    def wait_gather(rbuf, sem):
      pltpu.make_async_copy(x_hbm.at[pl.ds(0, CHUNK)], rbuf, sem).wait()

    def process(wbuf, rbuf, dbuf):
      def scale_group(g, carry2):
        wv = wbuf[pl.ds(g * 16, 16)]
        for t in range(16):
          w = wv[t]
          j = g * 16 + t
          for q in range(D // 16):
            rbuf[j, pl.ds(q * 16, 16)] = rbuf[j, pl.ds(q * 16, 16)] * w
        return carry2

      lax.fori_loop(0, CHUNK // 16, scale_group, 0)
      pltpu.sync_copy(rbuf, acc_sh.at[dbuf], add=True)

    for b in range(NBUF):
      idx(b, srcb[b], dstb[b], ewb[b])
      start_gather(srcb[b], rb[b], sems[b])

    def body(q, carry):
      i0 = NBUF * q
      for b in range(NBUF):
        wait_gather(rb[b], sems[b])
        process(ewb[b], rb[b], dstb[b])
        idx(jnp.minimum(i0 + b + NBUF, NCHUNKS - 1), srcb[b], dstb[b], ewb[b])
        start_gather(srcb[b], rb[b], sems[b])
      return carry

    lax.fori_loop(0, NCHUNKS // NBUF, body, 0)
    for b in range(NBUF):
      wait_gather(rb[b], sems[b])

    plsc.subcore_barrier()
    pltpu.sync_copy(acc_sh.at[pl.ds(s * ROWS_PER_TILE, ROWS_PER_TILE)],
                    out_hbm.at[c, pl.ds(s * ROWS_PER_TILE, ROWS_PER_TILE)])

  return spmm(x, src, dst, ew, zeros)


def _tc_combine(agg, x, W, W_self, b):
  def kern(agg_ref, x_ref, w_ref, ws_ref, b_ref, out_ref, stats_ref,
           sum_acc, sq_acc):
    i = pl.program_id(0)
    a = agg_ref[0] + agg_ref[1]
    y = (lax.dot(a, w_ref[...], precision=lax.Precision.HIGHEST)
         + lax.dot(x_ref[...], ws_ref[...], precision=lax.Precision.HIGHEST)
         + b_ref[...])
    out_ref[...] = y

    @pl.when(i == 0)
    def _():
      sum_acc[...] = jnp.zeros_like(sum_acc)
      sq_acc[...] = jnp.zeros_like(sq_acc)

    sum_acc[...] += jnp.sum(y, axis=0, keepdims=True)
    sq_acc[...] += jnp.sum(y * y, axis=0, keepdims=True)

    @pl.when(i == N_BLK - 1)
    def _():
      stats_ref[0:1, :] = sum_acc[...]
      stats_ref[1:2, :] = sq_acc[...]

  return pl.pallas_call(
      kern,
      grid=(N_BLK,),
      in_specs=[
          pl.BlockSpec((NC, ROW_BLK, D), lambda i: (0, i, 0)),
          pl.BlockSpec((ROW_BLK, D), lambda i: (i, 0)),
          pl.BlockSpec((D, D), lambda i: (0, 0)),
          pl.BlockSpec((D, D), lambda i: (0, 0)),
          pl.BlockSpec((1, D), lambda i: (0, 0)),
      ],
      out_specs=[
          pl.BlockSpec((ROW_BLK, D), lambda i: (i, 0)),
          pl.BlockSpec((2, D), lambda i: (0, 0)),
      ],
      out_shape=[
          jax.ShapeDtypeStruct((N_NODES, D), jnp.float32),
          jax.ShapeDtypeStruct((2, D), jnp.float32),
      ],
      scratch_shapes=[
          pltpu.VMEM((1, D), jnp.float32),
          pltpu.VMEM((1, D), jnp.float32),
      ],
  )(agg, x, W, W_self, b)


def _tc_batchnorm(out_pre, stats, gamma, beta):
  def kern(y_ref, st_ref, g_ref, bt_ref, o_ref):
    mean = st_ref[0:1, :] * (1.0 / N_NODES)
    var = st_ref[1:2, :] * (1.0 / N_NODES) - mean * mean
    inv = lax.rsqrt(var + 1e-5) * g_ref[...]
    o_ref[...] = (y_ref[...] - mean) * inv + bt_ref[...]

  return pl.pallas_call(
      kern,
      grid=(N_BLK,),
      in_specs=[
          pl.BlockSpec((ROW_BLK, D), lambda i: (i, 0)),
          pl.BlockSpec((2, D), lambda i: (0, 0)),
          pl.BlockSpec((1, D), lambda i: (0, 0)),
          pl.BlockSpec((1, D), lambda i: (0, 0)),
      ],
      out_specs=pl.BlockSpec((ROW_BLK, D), lambda i: (i, 0)),
      out_shape=jax.ShapeDtypeStruct((N_NODES, D), jnp.float32),
  )(out_pre, stats, gamma, beta)


def kernel(x, edge_index, edge_weight, W, W_self, b, bn_gamma, bn_beta):
  ei = edge_index.astype(jnp.int32)
  pad = E_PAD - N_EDGES
  src = jnp.concatenate([ei[0], jnp.zeros((pad,), jnp.int32)])
  pad_dst = N_NODES + (jnp.arange(pad, dtype=jnp.int32) % (NPAD - N_NODES))
  dst = jnp.concatenate([ei[1], pad_dst])
  edge_weight = jnp.concatenate([edge_weight, jnp.zeros((pad,), jnp.float32)])
  zeros = jnp.zeros((NPAD, D), jnp.float32)
  agg = _sc_edge_aggregate(x, src, dst, edge_weight, zeros)
  out_pre, stats = _tc_combine(agg, x, W, W_self, b.reshape(1, D))
  return _tc_batchnorm(out_pre, stats, bn_gamma.reshape(1, D),
                       bn_beta.reshape(1, D))


# speedup vs baseline: 2.2703x; 2.2687x over previous
import functools

import jax
import jax.numpy as jnp
from jax import lax
from jax.experimental import pallas as pl
from jax.experimental.pallas import tpu as pltpu
from jax.experimental.pallas import tpu_sc as plsc

N_NODES = 10000
D = 128
N_EDGES = 320000

NC = 2
NS = 16
NW = NC * NS
EPW = N_EDGES // NW
CHUNK = 80
NCHUNKS = EPW // CHUNK
NPAD = 10240
ROWS_PER_TILE = NPAD // NS

ROW_BLK = 1000
N_BLK = N_NODES // ROW_BLK


def _sc_edge_aggregate(x, src, dst, ew, zeros):
  mesh = plsc.VectorSubcoreMesh(core_axis_name="c", subcore_axis_name="s")

  @functools.partial(
      pl.kernel,
      mesh=mesh,
      out_type=jax.ShapeDtypeStruct((NC, NPAD, D), jnp.float32),
      scratch_types=[
          pltpu.VMEM((CHUNK,), jnp.int32),
          pltpu.VMEM((CHUNK,), jnp.int32),
          pltpu.VMEM((CHUNK,), jnp.int32),
          pltpu.VMEM((CHUNK,), jnp.int32),
          pltpu.VMEM((CHUNK,), jnp.float32),
          pltpu.VMEM((CHUNK,), jnp.float32),
          pltpu.VMEM((CHUNK, D), jnp.float32),
          pltpu.VMEM((CHUNK, D), jnp.float32),
          pltpu.VMEM_SHARED((NPAD, D), jnp.float32),
          pltpu.SemaphoreType.DMA,
          pltpu.SemaphoreType.DMA,
          pltpu.SemaphoreType.DMA,
          pltpu.SemaphoreType.DMA,
      ],
  )
  def spmm(x_hbm, src_hbm, dst_hbm, ew_hbm, z_hbm, out_hbm,
           src0, src1, dst0, dst1, ew0, ew1, r0, r1, acc_sh,
           semi0, semi1, sem0, sem1):
    c = lax.axis_index("c")
    s = lax.axis_index("s")
    wid = c * NS + s

    pltpu.sync_copy(z_hbm.at[pl.ds(s * ROWS_PER_TILE, ROWS_PER_TILE)],
                    acc_sh.at[pl.ds(s * ROWS_PER_TILE, ROWS_PER_TILE)])
    plsc.subcore_barrier()

    def start_idx(i, sbuf, dbuf, wbuf, sem):
      base = wid * EPW + i * CHUNK
      pltpu.async_copy(src_hbm.at[pl.ds(base, CHUNK)], sbuf, sem)
      pltpu.async_copy(dst_hbm.at[pl.ds(base, CHUNK)], dbuf, sem)
      pltpu.async_copy(ew_hbm.at[pl.ds(base, CHUNK)], wbuf, sem)

    def wait_idx(sbuf, dbuf, wbuf, sem):
      pltpu.make_async_copy(src_hbm.at[pl.ds(0, CHUNK)], sbuf, sem).wait()
      pltpu.make_async_copy(dst_hbm.at[pl.ds(0, CHUNK)], dbuf, sem).wait()
      pltpu.make_async_copy(ew_hbm.at[pl.ds(0, CHUNK)], wbuf, sem).wait()

    def start_gather(sbuf, rbuf, sem):
      pltpu.async_copy(x_hbm.at[sbuf], rbuf, sem)

    def wait_gather(rbuf, sem):
      pltpu.make_async_copy(x_hbm.at[pl.ds(0, CHUNK)], rbuf, sem).wait()

    def process(wbuf, rbuf, dbuf):
      def scale_group(g, carry2):
        wv = wbuf[pl.ds(g * 16, 16)]
        for t in range(16):
          w = wv[t]
          j = g * 16 + t
          for q in range(D // 16):
            rbuf[j, pl.ds(q * 16, 16)] = rbuf[j, pl.ds(q * 16, 16)] * w
        return carry2

      lax.fori_loop(0, CHUNK // 16, scale_group, 0)
      pltpu.sync_copy(rbuf, acc_sh.at[dbuf], add=True)

    start_idx(0, src0, dst0, ew0, semi0)
    start_idx(1, src1, dst1, ew1, semi1)
    wait_idx(src0, dst0, ew0, semi0)
    start_gather(src0, r0, sem0)

    def body(p, carry):
      i0 = 2 * p
      wait_idx(src1, dst1, ew1, semi1)
      start_gather(src1, r1, sem1)
      wait_gather(r0, sem0)
      process(ew0, r0, dst0)
      start_idx(i0 + 2, src0, dst0, ew0, semi0)
      wait_idx(src0, dst0, ew0, semi0)
      start_gather(src0, r0, sem0)
      wait_gather(r1, sem1)
      process(ew1, r1, dst1)
      start_idx(jnp.minimum(i0 + 3, NCHUNKS - 1), src1, dst1, ew1, semi1)
      return carry

    lax.fori_loop(0, (NCHUNKS - 1) // 2, body, 0)
    wait_gather(r0, sem0)
    process(ew0, r0, dst0)
    wait_idx(src1, dst1, ew1, semi1)

    plsc.subcore_barrier()
    pltpu.sync_copy(acc_sh.at[pl.ds(s * ROWS_PER_TILE, ROWS_PER_TILE)],
                    out_hbm.at[c, pl.ds(s * ROWS_PER_TILE, ROWS_PER_TILE)])

  return spmm(x, src, dst, ew, zeros)


def _tc_combine(agg, x, W, W_self, b):
  def kern(agg_ref, x_ref, w_ref, ws_ref, b_ref, out_ref, stats_ref,
           sum_acc, sq_acc):
    i = pl.program_id(0)
    a = agg_ref[0] + agg_ref[1]
    y = (lax.dot(a, w_ref[...], precision=lax.Precision.HIGHEST)
         + lax.dot(x_ref[...], ws_ref[...], precision=lax.Precision.HIGHEST)
         + b_ref[...])
    out_ref[...] = y

    @pl.when(i == 0)
    def _():
      sum_acc[...] = jnp.zeros_like(sum_acc)
      sq_acc[...] = jnp.zeros_like(sq_acc)

    sum_acc[...] += jnp.sum(y, axis=0, keepdims=True)
    sq_acc[...] += jnp.sum(y * y, axis=0, keepdims=True)

    @pl.when(i == N_BLK - 1)
    def _():
      stats_ref[0:1, :] = sum_acc[...]
      stats_ref[1:2, :] = sq_acc[...]

  return pl.pallas_call(
      kern,
      grid=(N_BLK,),
      in_specs=[
          pl.BlockSpec((NC, ROW_BLK, D), lambda i: (0, i, 0)),
          pl.BlockSpec((ROW_BLK, D), lambda i: (i, 0)),
          pl.BlockSpec((D, D), lambda i: (0, 0)),
          pl.BlockSpec((D, D), lambda i: (0, 0)),
          pl.BlockSpec((1, D), lambda i: (0, 0)),
      ],
      out_specs=[
          pl.BlockSpec((ROW_BLK, D), lambda i: (i, 0)),
          pl.BlockSpec((2, D), lambda i: (0, 0)),
      ],
      out_shape=[
          jax.ShapeDtypeStruct((N_NODES, D), jnp.float32),
          jax.ShapeDtypeStruct((2, D), jnp.float32),
      ],
      scratch_shapes=[
          pltpu.VMEM((1, D), jnp.float32),
          pltpu.VMEM((1, D), jnp.float32),
      ],
  )(agg, x, W, W_self, b)


def _tc_batchnorm(out_pre, stats, gamma, beta):
  def kern(y_ref, st_ref, g_ref, bt_ref, o_ref):
    mean = st_ref[0:1, :] * (1.0 / N_NODES)
    var = st_ref[1:2, :] * (1.0 / N_NODES) - mean * mean
    inv = lax.rsqrt(var + 1e-5) * g_ref[...]
    o_ref[...] = (y_ref[...] - mean) * inv + bt_ref[...]

  return pl.pallas_call(
      kern,
      grid=(N_BLK,),
      in_specs=[
          pl.BlockSpec((ROW_BLK, D), lambda i: (i, 0)),
          pl.BlockSpec((2, D), lambda i: (0, 0)),
          pl.BlockSpec((1, D), lambda i: (0, 0)),
          pl.BlockSpec((1, D), lambda i: (0, 0)),
      ],
      out_specs=pl.BlockSpec((ROW_BLK, D), lambda i: (i, 0)),
      out_shape=jax.ShapeDtypeStruct((N_NODES, D), jnp.float32),
  )(out_pre, stats, gamma, beta)


def kernel(x, edge_index, edge_weight, W, W_self, b, bn_gamma, bn_beta):
  ei = edge_index.astype(jnp.int32)
  src = ei[0]
  dst = ei[1]
  zeros = jnp.zeros((NPAD, D), jnp.float32)
  agg = _sc_edge_aggregate(x, src, dst, edge_weight, zeros)
  out_pre, stats = _tc_combine(agg, x, W, W_self, b.reshape(1, D))
  return _tc_batchnorm(out_pre, stats, bn_gamma.reshape(1, D),
                       bn_beta.reshape(1, D))


# speedup vs baseline: 2.3602x; 1.0396x over previous
import functools

import jax
import jax.numpy as jnp
from jax import lax
from jax.experimental import pallas as pl
from jax.experimental.pallas import tpu as pltpu
from jax.experimental.pallas import tpu_sc as plsc

N_NODES = 10000
D = 128
N_EDGES = 320000

NC = 2
NS = 16
NW = NC * NS
EPW = N_EDGES // NW
CHUNK = 80
NCHUNKS = EPW // CHUNK
NPAD = 10240
ROWS_PER_TILE = NPAD // NS

ROW_BLK = 2000
N_BLK = N_NODES // ROW_BLK


def _sc_edge_aggregate(x, src, dst, ew, zeros):
  mesh = plsc.VectorSubcoreMesh(core_axis_name="c", subcore_axis_name="s")

  @functools.partial(
      pl.kernel,
      mesh=mesh,
      out_type=jax.ShapeDtypeStruct((NC, NPAD, D), jnp.float32),
      scratch_types=[
          pltpu.VMEM((CHUNK,), jnp.int32),
          pltpu.VMEM((CHUNK,), jnp.int32),
          pltpu.VMEM((CHUNK,), jnp.int32),
          pltpu.VMEM((CHUNK,), jnp.int32),
          pltpu.VMEM((CHUNK,), jnp.float32),
          pltpu.VMEM((CHUNK,), jnp.float32),
          pltpu.VMEM((CHUNK, D), jnp.float32),
          pltpu.VMEM((CHUNK, D), jnp.float32),
          pltpu.VMEM_SHARED((NPAD, D), jnp.float32),
          pltpu.SemaphoreType.DMA,
          pltpu.SemaphoreType.DMA,
          pltpu.SemaphoreType.DMA,
          pltpu.SemaphoreType.DMA,
      ],
  )
  def spmm(x_hbm, src_hbm, dst_hbm, ew_hbm, z_hbm, out_hbm,
           src0, src1, dst0, dst1, ew0, ew1, r0, r1, acc_sh,
           semi0, semi1, sem0, sem1):
    c = lax.axis_index("c")
    s = lax.axis_index("s")
    wid = c * NS + s

    pltpu.sync_copy(z_hbm.at[pl.ds(s * ROWS_PER_TILE, ROWS_PER_TILE)],
                    acc_sh.at[pl.ds(s * ROWS_PER_TILE, ROWS_PER_TILE)])
    plsc.subcore_barrier()

    def start_idx(i, sbuf, dbuf, wbuf, sem):
      base = wid * EPW + i * CHUNK
      pltpu.async_copy(src_hbm.at[pl.ds(base, CHUNK)], sbuf, sem)
      pltpu.async_copy(dst_hbm.at[pl.ds(base, CHUNK)], dbuf, sem)
      pltpu.async_copy(ew_hbm.at[pl.ds(base, CHUNK)], wbuf, sem)

    def wait_idx(sbuf, dbuf, wbuf, sem):
      pltpu.make_async_copy(src_hbm.at[pl.ds(0, CHUNK)], sbuf, sem).wait()
      pltpu.make_async_copy(dst_hbm.at[pl.ds(0, CHUNK)], dbuf, sem).wait()
      pltpu.make_async_copy(ew_hbm.at[pl.ds(0, CHUNK)], wbuf, sem).wait()

    def start_gather(sbuf, rbuf, sem):
      pltpu.async_copy(x_hbm.at[sbuf], rbuf, sem)

    def wait_gather(rbuf, sem):
      pltpu.make_async_copy(x_hbm.at[pl.ds(0, CHUNK)], rbuf, sem).wait()

    def process(wbuf, rbuf, dbuf):
      def scale_group(g, carry2):
        wv = wbuf[pl.ds(g * 16, 16)]
        for t in range(16):
          w = wv[t]
          j = g * 16 + t
          for q in range(D // 16):
            rbuf[j, pl.ds(q * 16, 16)] = rbuf[j, pl.ds(q * 16, 16)] * w
        return carry2

      lax.fori_loop(0, CHUNK // 16, scale_group, 0)
      pltpu.sync_copy(rbuf, acc_sh.at[dbuf], add=True)

    start_idx(0, src0, dst0, ew0, semi0)
    start_idx(1, src1, dst1, ew1, semi1)
    wait_idx(src0, dst0, ew0, semi0)
    start_gather(src0, r0, sem0)

    def body(p, carry):
      i0 = 2 * p
      wait_idx(src1, dst1, ew1, semi1)
      start_gather(src1, r1, sem1)
      wait_gather(r0, sem0)
      process(ew0, r0, dst0)
      start_idx(i0 + 2, src0, dst0, ew0, semi0)
      wait_idx(src0, dst0, ew0, semi0)
      start_gather(src0, r0, sem0)
      wait_gather(r1, sem1)
      process(ew1, r1, dst1)
      start_idx(jnp.minimum(i0 + 3, NCHUNKS - 1), src1, dst1, ew1, semi1)
      return carry

    lax.fori_loop(0, (NCHUNKS - 1) // 2, body, 0)
    wait_gather(r0, sem0)
    process(ew0, r0, dst0)
    wait_idx(src1, dst1, ew1, semi1)

    plsc.subcore_barrier()
    pltpu.sync_copy(acc_sh.at[pl.ds(s * ROWS_PER_TILE, ROWS_PER_TILE)],
                    out_hbm.at[c, pl.ds(s * ROWS_PER_TILE, ROWS_PER_TILE)])

  return spmm(x, src, dst, ew, zeros)


def _tc_combine(agg, x, W, W_self, b):
  def kern(agg_ref, x_ref, w_ref, ws_ref, b_ref, out_ref, stats_ref,
           sum_acc, sq_acc):
    i = pl.program_id(0)
    a = agg_ref[0] + agg_ref[1]
    y = (lax.dot(a, w_ref[...], precision=lax.Precision.HIGHEST)
         + lax.dot(x_ref[...], ws_ref[...], precision=lax.Precision.HIGHEST)
         + b_ref[...])
    out_ref[...] = y

    @pl.when(i == 0)
    def _():
      sum_acc[...] = jnp.zeros_like(sum_acc)
      sq_acc[...] = jnp.zeros_like(sq_acc)

    sum_acc[...] += jnp.sum(y, axis=0, keepdims=True)
    sq_acc[...] += jnp.sum(y * y, axis=0, keepdims=True)

    @pl.when(i == N_BLK - 1)
    def _():
      stats_ref[0:1, :] = sum_acc[...]
      stats_ref[1:2, :] = sq_acc[...]

  return pl.pallas_call(
      kern,
      grid=(N_BLK,),
      in_specs=[
          pl.BlockSpec((NC, ROW_BLK, D), lambda i: (0, i, 0)),
          pl.BlockSpec((ROW_BLK, D), lambda i: (i, 0)),
          pl.BlockSpec((D, D), lambda i: (0, 0)),
          pl.BlockSpec((D, D), lambda i: (0, 0)),
          pl.BlockSpec((1, D), lambda i: (0, 0)),
      ],
      out_specs=[
          pl.BlockSpec((ROW_BLK, D), lambda i: (i, 0)),
          pl.BlockSpec((2, D), lambda i: (0, 0)),
      ],
      out_shape=[
          jax.ShapeDtypeStruct((N_NODES, D), jnp.float32),
          jax.ShapeDtypeStruct((2, D), jnp.float32),
      ],
      scratch_shapes=[
          pltpu.VMEM((1, D), jnp.float32),
          pltpu.VMEM((1, D), jnp.float32),
      ],
  )(agg, x, W, W_self, b)


def _tc_batchnorm(out_pre, stats, gamma, beta):
  def kern(y_ref, st_ref, g_ref, bt_ref, o_ref):
    mean = st_ref[0:1, :] * (1.0 / N_NODES)
    var = st_ref[1:2, :] * (1.0 / N_NODES) - mean * mean
    inv = lax.rsqrt(var + 1e-5) * g_ref[...]
    o_ref[...] = (y_ref[...] - mean) * inv + bt_ref[...]

  return pl.pallas_call(
      kern,
      grid=(N_BLK,),
      in_specs=[
          pl.BlockSpec((ROW_BLK, D), lambda i: (i, 0)),
          pl.BlockSpec((2, D), lambda i: (0, 0)),
          pl.BlockSpec((1, D), lambda i: (0, 0)),
          pl.BlockSpec((1, D), lambda i: (0, 0)),
      ],
      out_specs=pl.BlockSpec((ROW_BLK, D), lambda i: (i, 0)),
      out_shape=jax.ShapeDtypeStruct((N_NODES, D), jnp.float32),
  )(out_pre, stats, gamma, beta)


def kernel(x, edge_index, edge_weight, W, W_self, b, bn_gamma, bn_beta):
  ei = edge_index.astype(jnp.int32)
  src = ei[0]
  dst = ei[1]
  zeros = jnp.zeros((NPAD, D), jnp.float32)
  agg = _sc_edge_aggregate(x, src, dst, edge_weight, zeros)
  out_pre, stats = _tc_combine(agg, x, W, W_self, b.reshape(1, D))
  return _tc_batchnorm(out_pre, stats, bn_gamma.reshape(1, D),
                       bn_beta.reshape(1, D))
